# Initial kernel scaffold; baseline (speedup 1.0000x reference)
import jax
import jax.numpy as jnp
from jax.experimental import pallas as pl


def kernel(postion_ids, table):
    raise NotImplementedError("write your pallas kernel here")


# speedup vs baseline: 3.8290x; 3.8290x over previous
import jax
import jax.numpy as jnp
from jax.experimental import pallas as pl

D_MODEL = 1024
NUM_EMB = 4
PADDING_IDX = 3

_ROWS = 512


def _body(ids_ref, table_ref, out_ref):
    ids = ids_ref[0, 0, :]
    idb = ids[:, None]
    t0 = table_ref[0:1, :]
    t1 = table_ref[1:2, :]
    t2 = table_ref[2:3, :]
    out = jnp.where(
        idb == 0, t0,
        jnp.where(idb == 1, t1,
                  jnp.where(idb == 2, t2, jnp.float32(0.0))))
    out_ref[0, :, :] = out


def kernel(postion_ids, table):
    B, S = postion_ids.shape
    total = B * S
    nb = total // _ROWS
    ids3 = postion_ids.reshape(nb, 1, _ROWS).astype(jnp.int32)
    out = pl.pallas_call(
        _body,
        grid=(nb,),
        in_specs=[
            pl.BlockSpec((1, 1, _ROWS), lambda g: (g, 0, 0)),
            pl.BlockSpec((NUM_EMB, D_MODEL), lambda g: (0, 0)),
        ],
        out_specs=pl.BlockSpec((1, _ROWS, D_MODEL), lambda g: (g, 0, 0)),
        out_shape=jax.ShapeDtypeStruct((nb, _ROWS, D_MODEL), jnp.float32),
    )(ids3, table)
    return out.reshape(B, S, D_MODEL)
